# Initial kernel scaffold; baseline (speedup 1.0000x reference)
import jax
import jax.numpy as jnp
from jax.experimental import pallas as pl


def kernel(x, route_id, pi_W1, pi_b1, pi_g1, pi_be1, pi_W2, pi_b2, pi_g2, pi_be2, l0_Ws, l0_bs, l0_We, l0_be, l1_Ws, l1_bs, l1_We, l1_be, po_W, po_b):
    raise NotImplementedError("write your pallas kernel here")


# speedup vs baseline: 9.5274x; 9.5274x over previous
import jax
import jax.numpy as jnp
from jax.experimental import pallas as pl
from jax.experimental.pallas import tpu as pltpu

D = 1024
H = 128
E = 8
TB = 512
S = 2048


def _ln(x, g, b, eps=1e-5):
    m = jnp.mean(x, axis=-1, keepdims=True)
    v = jnp.mean((x - m) ** 2, axis=-1, keepdims=True)
    return (x - m) * jax.lax.rsqrt(v + eps) * g + b


def _dot_t(a, w):
    return jax.lax.dot_general(a, w, (((1,), (1,)), ((), ())),
                               preferred_element_type=jnp.float32)


def _fused_kernel(rid_ref, x_ref,
                  pi_W1_ref, pi_b1_ref, pi_g1_ref, pi_be1_ref,
                  pi_W2_ref, pi_b2_ref, pi_g2_ref, pi_be2_ref,
                  l0_Ws_ref, l0_bs_ref, l0_We_ref, l0_be_ref,
                  l1_Ws_ref, l1_bs_ref, l1_We_ref, l1_be_ref,
                  po_W_ref, po_b_ref, o_ref):
    x = x_ref[0]
    h = _dot_t(x, pi_W1_ref[...]) + pi_b1_ref[...]
    h = jax.nn.relu(_ln(h, pi_g1_ref[...], pi_be1_ref[...]))
    res = _dot_t(h, pi_W2_ref[...]) + pi_b2_ref[...]
    y = jax.nn.relu(_ln(res, pi_g2_ref[...], pi_be2_ref[...]) + x)
    sh = _dot_t(y, l0_Ws_ref[...]) + l0_bs_ref[...]
    rt = _dot_t(y, l0_We_ref[0]) + l0_be_ref[0]
    y = jax.nn.relu(sh + rt + y)
    sh = _dot_t(y, l1_Ws_ref[...]) + l1_bs_ref[...]
    rt = _dot_t(y, l1_We_ref[0]) + l1_be_ref[0]
    y = jax.nn.relu(sh + rt + y)
    o_ref[0] = _dot_t(y, po_W_ref[...]) + po_b_ref[...]


def kernel(x, route_id, pi_W1, pi_b1, pi_g1, pi_be1, pi_W2, pi_b2, pi_g2,
           pi_be2, l0_Ws, l0_bs, l0_We, l0_be, l1_Ws, l1_bs, l1_We, l1_be,
           po_W, po_b):
    B = x.shape[0]
    rid = route_id.astype(jnp.int32)
    const2 = lambda s, j, r: (0, 0)
    expw = lambda s, j, r: (r[s], 0, 0)
    expb = lambda s, j, r: (r[s], 0, 0)
    row = lambda a: a.reshape(1, -1)

    grid = (B, S // TB)
    out = pl.pallas_call(
        _fused_kernel,
        grid_spec=pltpu.PrefetchScalarGridSpec(
            num_scalar_prefetch=1,
            grid=grid,
            in_specs=[
                pl.BlockSpec((1, TB, D), lambda s, j, r: (s, j, 0)),
                pl.BlockSpec((H, D), const2),
                pl.BlockSpec((1, H), const2),
                pl.BlockSpec((1, H), const2),
                pl.BlockSpec((1, H), const2),
                pl.BlockSpec((D, H), const2),
                pl.BlockSpec((1, D), const2),
                pl.BlockSpec((1, D), const2),
                pl.BlockSpec((1, D), const2),
                pl.BlockSpec((D, D), const2),
                pl.BlockSpec((1, D), const2),
                pl.BlockSpec((1, D, D), expw),
                pl.BlockSpec((1, 1, D), expb),
                pl.BlockSpec((D, D), const2),
                pl.BlockSpec((1, D), const2),
                pl.BlockSpec((1, D, D), expw),
                pl.BlockSpec((1, 1, D), expb),
                pl.BlockSpec((D, D), const2),
                pl.BlockSpec((1, D), const2),
            ],
            out_specs=pl.BlockSpec((1, TB, D), lambda s, j, r: (s, j, 0)),
        ),
        out_shape=jax.ShapeDtypeStruct((B, S, D), x.dtype),
    )(rid, x,
      pi_W1, row(pi_b1), row(pi_g1), row(pi_be1),
      pi_W2, row(pi_b2), row(pi_g2), row(pi_be2),
      l0_Ws, row(l0_bs), l0_We, l0_be.reshape(E, 1, D),
      l1_Ws, row(l1_bs), l1_We, l1_be.reshape(E, 1, D),
      po_W, row(po_b))
    return out
